# baseline (device time: 11848 ns/iter reference)
import jax
import jax.numpy as jnp
from jax import lax
from jax.experimental import pallas as pl
from jax.experimental.pallas import tpu as pltpu

X_SIZE = 2
ROW_CHUNK = 256


def kernel(x):
    m_per, n_per = x.shape
    m_global = X_SIZE * m_per
    n_chunks = m_per // ROW_CHUNK

    def body(x_ref, out_ref, comm_ref, send_sem, recv_sem):
        my_x = lax.axis_index("x")
        my_y = lax.axis_index("y")
        peer = (1 - my_x, my_y)

        comm_ref[0, :, :] = jnp.zeros_like(comm_ref[0])
        for k in range(n_chunks):
            comm_ref[0, :, :] += jnp.sum(
                x_ref[pl.ds(k * ROW_CHUNK, ROW_CHUNK), :],
                axis=0, keepdims=True,
            )

        rdma = pltpu.make_async_remote_copy(
            src_ref=comm_ref.at[0],
            dst_ref=comm_ref.at[1],
            send_sem=send_sem,
            recv_sem=recv_sem,
            device_id=peer,
            device_id_type=pl.DeviceIdType.MESH,
        )
        rdma.start()
        rdma.wait()

        out_ref[:, :] = (comm_ref[0, :, :] + comm_ref[1, :, :]) * (
            1.0 / m_global
        )

    return pl.pallas_call(
        body,
        out_shape=jax.ShapeDtypeStruct((1, n_per), x.dtype),
        in_specs=[pl.BlockSpec(memory_space=pltpu.VMEM)],
        out_specs=pl.BlockSpec(memory_space=pltpu.VMEM),
        scratch_shapes=[
            pltpu.VMEM((2, 1, n_per), x.dtype),
            pltpu.SemaphoreType.DMA,
            pltpu.SemaphoreType.DMA,
        ],
    )(x)


# device time: 8224 ns/iter; 1.4407x vs baseline; 1.4407x over previous
import jax
import jax.numpy as jnp
from jax import lax
from jax.experimental import pallas as pl
from jax.experimental.pallas import tpu as pltpu

X_SIZE = 2
ROW_CHUNK = 256


def kernel(x):
    m_per, n_per = x.shape
    m_global = X_SIZE * m_per
    n_chunks = m_per // ROW_CHUNK

    def body(x_ref, out_ref, comm_ref, send_sem, recv_sem):
        my_x = lax.axis_index("x")
        my_y = lax.axis_index("y")
        peer = (1 - my_x, my_y)

        barrier_sem = pltpu.get_barrier_semaphore()
        pl.semaphore_signal(
            barrier_sem, inc=1, device_id=(my_x, my_y),
            device_id_type=pl.DeviceIdType.MESH,
        )
        pl.semaphore_wait(barrier_sem, 1)

        comm_ref[0, :, :] = jnp.zeros_like(comm_ref[0])
        for k in range(n_chunks):
            comm_ref[0, :, :] += jnp.sum(
                x_ref[pl.ds(k * ROW_CHUNK, ROW_CHUNK), :],
                axis=0, keepdims=True,
            )

        rdma = pltpu.make_async_remote_copy(
            src_ref=comm_ref.at[0],
            dst_ref=comm_ref.at[1],
            send_sem=send_sem,
            recv_sem=recv_sem,
            device_id=peer,
            device_id_type=pl.DeviceIdType.MESH,
        )
        rdma.start()
        rdma.wait()

        out_ref[:, :] = (comm_ref[0, :, :] + comm_ref[1, :, :]) * (
            1.0 / m_global
        )

    return pl.pallas_call(
        body,
        out_shape=jax.ShapeDtypeStruct((1, n_per), x.dtype),
        in_specs=[pl.BlockSpec(memory_space=pltpu.VMEM)],
        out_specs=pl.BlockSpec(memory_space=pltpu.VMEM),
        scratch_shapes=[
            pltpu.VMEM((2, 1, n_per), x.dtype),
            pltpu.SemaphoreType.DMA,
            pltpu.SemaphoreType.DMA,
        ],
        compiler_params=pltpu.CompilerParams(collective_id=0),
    )(x)


# device time: 8072 ns/iter; 1.4678x vs baseline; 1.0188x over previous
import jax
import jax.numpy as jnp
from jax import lax
from jax.experimental import pallas as pl
from jax.experimental.pallas import tpu as pltpu

X_SIZE = 2
ROW_CHUNK = 256


def kernel(x):
    m_per, n_per = x.shape
    m_global = X_SIZE * m_per
    n_chunks = m_per // ROW_CHUNK

    def body(x_ref, out_ref, comm_ref, send_sem, recv_sem):
        my_x = lax.axis_index("x")
        my_y = lax.axis_index("y")
        peer = (1 - my_x, my_y)

        barrier_sem = pltpu.get_barrier_semaphore()
        pl.semaphore_signal(barrier_sem, inc=1)
        pl.semaphore_wait(barrier_sem, 1)

        comm_ref[0, :, :] = jnp.zeros_like(comm_ref[0])
        for k in range(n_chunks):
            comm_ref[0, :, :] += jnp.sum(
                x_ref[pl.ds(k * ROW_CHUNK, ROW_CHUNK), :],
                axis=0, keepdims=True,
            )

        rdma = pltpu.make_async_remote_copy(
            src_ref=comm_ref.at[0],
            dst_ref=comm_ref.at[1],
            send_sem=send_sem,
            recv_sem=recv_sem,
            device_id=peer,
            device_id_type=pl.DeviceIdType.MESH,
        )
        rdma.start()
        rdma.wait()

        out_ref[:, :] = (comm_ref[0, :, :] + comm_ref[1, :, :]) * (
            1.0 / m_global
        )

    return pl.pallas_call(
        body,
        out_shape=jax.ShapeDtypeStruct((1, n_per), x.dtype),
        in_specs=[pl.BlockSpec(memory_space=pltpu.VMEM)],
        out_specs=pl.BlockSpec(memory_space=pltpu.VMEM),
        scratch_shapes=[
            pltpu.VMEM((2, 1, n_per), x.dtype),
            pltpu.SemaphoreType.DMA,
            pltpu.SemaphoreType.DMA,
        ],
        compiler_params=pltpu.CompilerParams(collective_id=0),
    )(x)
